# baseline (speedup 1.0000x reference)
import functools

import jax
import jax.numpy as jnp
from jax.experimental import pallas as pl
from jax.experimental.pallas import tpu as pltpu

_CHUNK = 512
_NBUF = 4


def _router_kernel(x_hbm, w_ref, o_hbm, xbuf, obuf, in_sems, out_sems):
    n_chunks = x_hbm.shape[0] // _CHUNK

    def in_copy(i, slot):
        return pltpu.make_async_copy(
            x_hbm.at[pl.ds(i * _CHUNK, _CHUNK), :],
            xbuf.at[slot],
            in_sems.at[slot],
        )

    def out_copy(i, slot):
        return pltpu.make_async_copy(
            obuf.at[slot],
            o_hbm.at[pl.ds(i * _CHUNK, _CHUNK), :],
            out_sems.at[slot],
        )

    for s in range(min(_NBUF, n_chunks)):
        in_copy(s, s).start()

    for i in range(n_chunks):
        slot = i % _NBUF
        in_copy(i, slot).wait()
        if i >= _NBUF:
            out_copy(i - _NBUF, slot).wait()
        obuf[slot] = jax.lax.dot_general(
            xbuf[slot],
            w_ref[...],
            (((1,), (1,)), ((), ())),
            preferred_element_type=jnp.float32,
        )
        out_copy(i, slot).start()
        if i + _NBUF < n_chunks:
            in_copy(i + _NBUF, slot).start()

    for i in range(max(0, n_chunks - _NBUF), n_chunks):
        out_copy(i, i % _NBUF).wait()


@functools.partial(jax.jit, static_argnames=())
def kernel(x, gate_w):
    b, t, d = x.shape
    e = gate_w.shape[0]
    m = b * t
    x2 = x.reshape(m, d)

    out = pl.pallas_call(
        _router_kernel,
        in_specs=[
            pl.BlockSpec(memory_space=pl.ANY),
            pl.BlockSpec(memory_space=pltpu.VMEM),
        ],
        out_specs=pl.BlockSpec(memory_space=pl.ANY),
        out_shape=jax.ShapeDtypeStruct((m, e), jnp.float32),
        scratch_shapes=[
            pltpu.VMEM((_NBUF, _CHUNK, d), jnp.float32),
            pltpu.VMEM((_NBUF, _CHUNK, e), jnp.float32),
            pltpu.SemaphoreType.DMA((_NBUF,)),
            pltpu.SemaphoreType.DMA((_NBUF,)),
        ],
    )(x2, gate_w)
    return out.reshape(b, t, e)
